# Initial kernel scaffold; baseline (speedup 1.0000x reference)
import jax
import jax.numpy as jnp
from jax.experimental import pallas as pl


def kernel(op_gid, cbo, enc, op_table, ln_gamma, ln_beta):
    raise NotImplementedError("write your pallas kernel here")


# speedup vs baseline: 2.1970x; 2.1970x over previous
import functools

import jax
import jax.numpy as jnp
from jax import lax
from jax.experimental import pallas as pl
from jax.experimental.pallas import tpu as pltpu
from jax.experimental.pallas import tpu_sc as plsc

N = 100000
N_TYPES = 1000
D_EMB = 32
D_CBO = 48
D_ENC = 48
D_OUT = 128
L = 16
CH = 64
N_FULL = N // CH
TAIL = N - N_FULL * CH
TAIL_BASE = N_FULL * CH
NC = 2
NS = 16
NW = NC * NS
STEPS = (N_FULL + NW - 1) // NW
EPS = 1e-5
RSQRT_MAGIC = 0x5F3759DF
TS = (L + 1) * L


def _rsqrt(var):
    half = var * 0.5
    seed = jnp.full((L,), RSQRT_MAGIC, jnp.int32) - (
        plsc.bitcast(var, jnp.int32) >> 1)
    y = plsc.bitcast(seed, jnp.float32)
    y = y * (1.5 - half * y * y)
    y = y * (1.5 - half * y * y)
    y = y * (1.5 - half * y * y)
    return y


def _tree_sum(vals):
    while len(vals) > 1:
        vals = [a + b for a, b in zip(vals[::2], vals[1::2])]
    return vals[0]


def _make_sc_kernel():
    mesh = plsc.VectorSubcoreMesh(core_axis_name="c", subcore_axis_name="s")

    @functools.partial(
        pl.kernel,
        mesh=mesh,
        compiler_params=pltpu.CompilerParams(needs_layout_passes=False,
                                             use_tc_tiling_on_sc=True),
        out_type=jax.ShapeDtypeStruct((N, D_OUT), jnp.float32),
        scratch_types=[
            pltpu.VMEM((N_TYPES * D_EMB,), jnp.float32),
            pltpu.VMEM((CH,), jnp.int32),
            pltpu.VMEM((CH,), jnp.int32),
            pltpu.VMEM((CH, D_CBO), jnp.float32),
            pltpu.VMEM((CH, D_CBO), jnp.float32),
            pltpu.VMEM((CH, D_ENC), jnp.float32),
            pltpu.VMEM((CH, D_ENC), jnp.float32),
            pltpu.VMEM((CH, D_OUT), jnp.float32),
            pltpu.VMEM((CH, D_OUT), jnp.float32),
            pltpu.VMEM((TAIL,), jnp.int32),
            pltpu.VMEM((TAIL, D_CBO), jnp.float32),
            pltpu.VMEM((TAIL, D_ENC), jnp.float32),
            pltpu.VMEM((TAIL, D_OUT), jnp.float32),
            pltpu.VMEM(((CH // L) * 2 * (L + 1) * L,), jnp.float32),
            pltpu.VMEM((D_OUT,), jnp.float32),
            pltpu.VMEM((D_OUT,), jnp.float32),
            pltpu.SemaphoreType.DMA,
            pltpu.SemaphoreType.DMA,
            pltpu.SemaphoreType.DMA,
            pltpu.SemaphoreType.DMA,
        ],
    )
    def sc_embed(gid_h, cbo_h, enc_h, tab_h, gam_h, bet_h, out_h,
                 tab_v, idx_v0, idx_v1, cbo_v0, cbo_v1, enc_v0, enc_v1,
                 out_v0, out_v1,
                 idx_t, cbo_t, enc_t, out_t,
                 ts_v, gam_v, bet_v,
                 sem_in0, sem_in1, sem_out0, sem_out1):
        idx_b = (idx_v0, idx_v1)
        cbo_b = (cbo_v0, cbo_v1)
        enc_b = (enc_v0, enc_v1)
        out_b = (out_v0, out_v1)
        sem_in = (sem_in0, sem_in1)
        sem_out = (sem_out0, sem_out1)
        wid = lax.axis_index("s") * NC + lax.axis_index("c")
        pltpu.sync_copy(tab_h, tab_v)
        pltpu.sync_copy(gam_h, gam_v)
        pltpu.sync_copy(bet_h, bet_v)
        nvec = D_OUT // L
        gs = [gam_v[pl.ds(j * L, L)] for j in range(nvec)]
        bs = [bet_v[pl.ds(j * L, L)] for j in range(nvec)]
        iota16 = lax.iota(jnp.int32, L)

        def load_row(gsp, cbo, enc, r):
            xs = [plsc.load_gather(tab_v, [gsp + (v * L + iota16)])
                  for v in range(D_EMB // L)]
            xs += [cbo[r, pl.ds(v * L, L)]
                   for v in range(D_CBO // L)]
            xs += [enc[r, pl.ds(v * L, L)]
                   for v in range(D_ENC // L)]
            return xs

        def fused_pass(idx, cbo, enc, out, ch):
            @plsc.parallel_loop(0, ch // L)
            def block(b):
                r0 = b * L
                tbase = b * 2 * TS
                qbase = tbase + TS
                gid32 = idx[pl.ds(r0, L)] * D_EMB
                for i in range(L):
                    gsp = jnp.full((L,), gid32[i], jnp.int32)
                    xs = load_row(gsp, cbo, enc, r0 + i)
                    psum = _tree_sum(list(xs))
                    qsum = _tree_sum([x * x for x in xs])
                    ts_v[pl.ds(tbase + i * (L + 1), L)] = psum
                    ts_v[pl.ds(qbase + i * (L + 1), L)] = qsum
                sidx = iota16 * (L + 1)
                ssum = _tree_sum([plsc.load_gather(ts_v, [tbase + sidx + l])
                                  for l in range(L)])
                qsum = _tree_sum([plsc.load_gather(ts_v, [qbase + sidx + l])
                                  for l in range(L)])
                mean = ssum * (1.0 / D_OUT)
                var = qsum * (1.0 / D_OUT) - mean * mean + EPS
                rs = _rsqrt(var)
                nm = -(mean * rs)
                for i in range(L):
                    r = r0 + i
                    rs_i = jnp.full((L,), rs[i], jnp.float32)
                    nm_i = jnp.full((L,), nm[i], jnp.float32)
                    gsp = jnp.full((L,), gid32[i], jnp.int32)
                    xs = load_row(gsp, cbo, enc, r)
                    for j, x in enumerate(xs):
                        out[r, pl.ds(j * L, L)] = (
                            (x * rs_i + nm_i) * gs[j] + bs[j])

        def in_copies(b, base):
            return (
                pltpu.make_async_copy(
                    gid_h.at[pl.ds(base, CH)], idx_b[b], sem_in[b]),
                pltpu.make_async_copy(
                    cbo_h.at[pl.ds(base, CH)],
                    cbo_b[b], sem_in[b]),
                pltpu.make_async_copy(
                    enc_h.at[pl.ds(base, CH)],
                    enc_b[b], sem_in[b]),
            )

        def out_copy(b, base):
            return pltpu.make_async_copy(
                out_b[b], out_h.at[pl.ds(base, CH)],
                sem_out[b])

        def issue_in(b, base):
            for c in in_copies(b, base):
                c.start()

        def wait_in(b, base):
            for c in in_copies(b, base):
                c.wait()

        def half(b, cid):
            cid_next = cid + NW

            @pl.when(cid_next < N_FULL)
            def _():
                @pl.when(cid >= NW)
                def _():
                    out_copy(1 - b, (cid - NW) * CH).wait()

                issue_in(1 - b, cid_next * CH)

            @pl.when(cid < N_FULL)
            def _():
                base = cid * CH
                wait_in(b, base)
                fused_pass(idx_b[b], cbo_b[b], enc_b[b], out_b[b], CH)
                out_copy(b, base).start()

        issue_in(0, wid * CH)

        def pair_body(p, carry):
            cid0 = (p * 2) * NW + wid
            half(0, cid0)
            half(1, cid0 + NW)
            return carry

        npairs = (STEPS + 2) // 2
        lax.fori_loop(0, npairs, pair_body, 0)

        out_copy(0, 0).wait()
        out_copy(1, 0).wait()

        @pl.when(wid == NW - 1)
        def _():
            pltpu.sync_copy(gid_h.at[pl.ds(TAIL_BASE, TAIL)], idx_t)
            pltpu.sync_copy(
                cbo_h.at[pl.ds(TAIL_BASE, TAIL)], cbo_t)
            pltpu.sync_copy(
                enc_h.at[pl.ds(TAIL_BASE, TAIL)], enc_t)
            fused_pass(idx_t, cbo_t, enc_t, out_t, TAIL)
            pltpu.sync_copy(
                out_t, out_h.at[pl.ds(TAIL_BASE, TAIL)])

    return sc_embed


_SC_EMBED = _make_sc_kernel()


def kernel(op_gid, cbo, enc, op_table, ln_gamma, ln_beta):
    return _SC_EMBED(op_gid.astype(jnp.int32), cbo, enc,
                     op_table.reshape(-1), ln_gamma, ln_beta)
